# baseline (speedup 1.0000x reference)
import functools

import jax
import jax.numpy as jnp
from jax import lax
from jax.experimental import pallas as pl
from jax.experimental.pallas import tpu as pltpu
from jax.experimental.pallas import tpu_sc as plsc

_NC = 2
_NS = 16
_NW = _NC * _NS
_L = 16

_ROWS = 2
_E = 1_600_000
_DIL = 2
_OUT_E = _E // _DIL

_CC = 6_400
_OC = _CC // _DIL
_N_CHUNK = _E // _CC
_ROUNDS = -(-_N_CHUNK // _NW)
_NBUF = 4
_PDI = 3
_PD = 1

_mesh = plsc.VectorSubcoreMesh(
    core_axis_name="c", subcore_axis_name="s", num_cores=_NC, num_subcores=_NS
)


@functools.partial(
    pl.kernel,
    out_type=(
        jax.ShapeDtypeStruct((_ROWS, _OUT_E), jnp.int32),
        jax.ShapeDtypeStruct((1, _E), jnp.float32),
    ),
    mesh=_mesh,
    scratch_types=(
        [pltpu.VMEM((_ROWS, _CC), jnp.int32) for _ in range(_NBUF)]
        + [pltpu.VMEM((_ROWS, _OC), jnp.int32) for _ in range(_NBUF)]
        + [pltpu.VMEM((1, _CC), jnp.float32) for _ in range(_NBUF)]
        + [pltpu.SemaphoreType.DMA for _ in range(4 * _NBUF)]
    ),
    compiler_params=pltpu.CompilerParams(needs_layout_passes=False),
)
def _dilate(in_hbm, attr_hbm, out_hbm, attr_out_hbm, *bufs_and_sems):
    in_bufs = bufs_and_sems[:_NBUF]
    out_bufs = bufs_and_sems[_NBUF:2 * _NBUF]
    at_bufs = bufs_and_sems[2 * _NBUF:3 * _NBUF]
    sems = bufs_and_sems[3 * _NBUF:]
    sins = sems[:_NBUF]
    souts = sems[_NBUF:2 * _NBUF]
    sais = sems[2 * _NBUF:3 * _NBUF]
    saos = sems[3 * _NBUF:]

    cid = lax.axis_index("c")
    sid = lax.axis_index("s")
    wid = sid * _NC + cid

    lane = lax.iota(jnp.int32, _L)

    def chunk_id(t):
        return wid + t * _NW

    def valid(t):
        return chunk_id(t) < _N_CHUNK

    def in_pair(t, b):
        return (in_hbm.at[:, pl.ds(chunk_id(t) * _CC, _CC)], in_bufs[b], sins[b])

    def out_pair(t, b):
        return (out_bufs[b], out_hbm.at[:, pl.ds(chunk_id(t) * _OC, _OC)], souts[b])

    def attr_in_pair(t, b):
        return (attr_hbm.at[:, pl.ds(chunk_id(t) * _CC, _CC)], at_bufs[b], sais[b])

    def attr_out_pair(t, b):
        return (at_bufs[b], attr_out_hbm.at[:, pl.ds(chunk_id(t) * _CC, _CC)], saos[b])

    for p in range(_PDI):
        @pl.when(valid(p))
        def _(p=p):
            pltpu.async_copy(*in_pair(p, p % _NBUF))
            if p < _PD:
                pltpu.async_copy(*attr_in_pair(p, p % _NBUF))

    def round_body(t, b):
        bp = (b + _PD) % _NBUF

        @pl.when(valid(t + _PDI))
        def _():
            pltpu.async_copy(*in_pair(t + _PDI, (b + _PDI) % _NBUF))

        @pl.when(valid(t + _PD))
        def _():
            @pl.when(t + _PD >= _NBUF)
            def _():
                pltpu.make_async_copy(*attr_out_pair(t + _PD - _NBUF, bp)).wait()
            pltpu.async_copy(*attr_in_pair(t + _PD, bp))

        @pl.when(valid(t))
        def _():
            pltpu.make_async_copy(*attr_in_pair(t, b)).wait()
            pltpu.async_copy(*attr_out_pair(t, b))
            pltpu.make_async_copy(*in_pair(t, b)).wait()
            @pl.when(t >= _NBUF)
            def _():
                pltpu.make_async_copy(*out_pair(t - _NBUF, b)).wait()
            src = in_bufs[b]
            dst = out_bufs[b]
            for r in range(_ROWS):
                row_idx = jnp.full((_L,), r, jnp.int32)

                @plsc.parallel_loop(0, _OC // _L, unroll=4)
                def _(i):
                    col_idx = (lane + i * _L) * _DIL
                    dst[r, pl.ds(i * _L, _L)] = plsc.load_gather(
                        src, [row_idx, col_idx])

            pltpu.async_copy(*out_pair(t, b))

    def ring_body(tt, carry):
        for b in range(_NBUF):
            round_body(_NBUF * tt + b, b)
        return carry

    lax.fori_loop(0, _ROUNDS // _NBUF, ring_body, 0)

    for t in range(_ROUNDS):
        @pl.when(jnp.logical_and(valid(t), jnp.logical_not(valid(t + _NBUF))))
        def _(t=t):
            pltpu.make_async_copy(*out_pair(t, t % _NBUF)).wait()
            pltpu.make_async_copy(*attr_out_pair(t, t % _NBUF)).wait()


def kernel(edge_index, edge_attr):
    out, attr_out = _dilate(edge_index, edge_attr)
    return out, attr_out


# speedup vs baseline: 1.0277x; 1.0277x over previous
import functools

import jax
import jax.numpy as jnp
from jax import lax
from jax.experimental import pallas as pl
from jax.experimental.pallas import tpu as pltpu
from jax.experimental.pallas import tpu_sc as plsc

_NC = 2
_NS = 16
_NW = _NC * _NS
_L = 16

_ROWS = 2
_E = 1_600_000
_DIL = 2
_OUT_E = _E // _DIL

_CC = 6_400
_OC = _CC // _DIL
_N_CHUNK = _E // _CC
_ROUNDS = -(-_N_CHUNK // _NW)
_NBUF = 4
_PDI = 3
_PD = 2

_mesh = plsc.VectorSubcoreMesh(
    core_axis_name="c", subcore_axis_name="s", num_cores=_NC, num_subcores=_NS
)


@functools.partial(
    pl.kernel,
    out_type=(
        jax.ShapeDtypeStruct((_ROWS, _OUT_E), jnp.int32),
        jax.ShapeDtypeStruct((1, _E), jnp.float32),
    ),
    mesh=_mesh,
    scratch_types=(
        [pltpu.VMEM((_ROWS, _CC), jnp.int32) for _ in range(_NBUF)]
        + [pltpu.VMEM((_ROWS, _OC), jnp.int32) for _ in range(_NBUF)]
        + [pltpu.VMEM((1, _CC), jnp.float32) for _ in range(_NBUF)]
        + [pltpu.SemaphoreType.DMA for _ in range(4 * _NBUF)]
    ),
    compiler_params=pltpu.CompilerParams(needs_layout_passes=False),
)
def _dilate(in_hbm, attr_hbm, out_hbm, attr_out_hbm, *bufs_and_sems):
    in_bufs = bufs_and_sems[:_NBUF]
    out_bufs = bufs_and_sems[_NBUF:2 * _NBUF]
    at_bufs = bufs_and_sems[2 * _NBUF:3 * _NBUF]
    sems = bufs_and_sems[3 * _NBUF:]
    sins = sems[:_NBUF]
    souts = sems[_NBUF:2 * _NBUF]
    sais = sems[2 * _NBUF:3 * _NBUF]
    saos = sems[3 * _NBUF:]

    cid = lax.axis_index("c")
    sid = lax.axis_index("s")
    wid = sid * _NC + cid

    lane = lax.iota(jnp.int32, _L)

    def chunk_id(t):
        return wid + t * _NW

    def valid(t):
        return chunk_id(t) < _N_CHUNK

    def in_pair(t, b):
        return (in_hbm.at[:, pl.ds(chunk_id(t) * _CC, _CC)], in_bufs[b], sins[b])

    def out_pair(t, b):
        return (out_bufs[b], out_hbm.at[:, pl.ds(chunk_id(t) * _OC, _OC)], souts[b])

    def attr_in_pair(t, b):
        return (attr_hbm.at[:, pl.ds(chunk_id(t) * _CC, _CC)], at_bufs[b], sais[b])

    def attr_out_pair(t, b):
        return (at_bufs[b], attr_out_hbm.at[:, pl.ds(chunk_id(t) * _CC, _CC)], saos[b])

    for p in range(_PDI):
        @pl.when(valid(p))
        def _(p=p):
            pltpu.async_copy(*in_pair(p, p % _NBUF))
            if p < _PD:
                pltpu.async_copy(*attr_in_pair(p, p % _NBUF))

    def round_body(t, b):
        bp = (b + _PD) % _NBUF

        @pl.when(valid(t + _PDI))
        def _():
            pltpu.async_copy(*in_pair(t + _PDI, (b + _PDI) % _NBUF))

        @pl.when(valid(t + _PD))
        def _():
            @pl.when(t + _PD >= _NBUF)
            def _():
                pltpu.make_async_copy(*attr_out_pair(t + _PD - _NBUF, bp)).wait()
            pltpu.async_copy(*attr_in_pair(t + _PD, bp))

        @pl.when(valid(t))
        def _():
            pltpu.make_async_copy(*attr_in_pair(t, b)).wait()
            pltpu.async_copy(*attr_out_pair(t, b))
            pltpu.make_async_copy(*in_pair(t, b)).wait()
            @pl.when(t >= _NBUF)
            def _():
                pltpu.make_async_copy(*out_pair(t - _NBUF, b)).wait()
            src = in_bufs[b]
            dst = out_bufs[b]
            for r in range(_ROWS):
                row_idx = jnp.full((_L,), r, jnp.int32)

                @plsc.parallel_loop(0, _OC // _L, unroll=4)
                def _(i):
                    col_idx = (lane + i * _L) * _DIL
                    dst[r, pl.ds(i * _L, _L)] = plsc.load_gather(
                        src, [row_idx, col_idx])

            pltpu.async_copy(*out_pair(t, b))

    def ring_body(tt, carry):
        for b in range(_NBUF):
            round_body(_NBUF * tt + b, b)
        return carry

    lax.fori_loop(0, _ROUNDS // _NBUF, ring_body, 0)

    for t in range(_ROUNDS):
        @pl.when(jnp.logical_and(valid(t), jnp.logical_not(valid(t + _NBUF))))
        def _(t=t):
            pltpu.make_async_copy(*out_pair(t, t % _NBUF)).wait()
            pltpu.make_async_copy(*attr_out_pair(t, t % _NBUF)).wait()


def kernel(edge_index, edge_attr):
    out, attr_out = _dilate(edge_index, edge_attr)
    return out, attr_out
